# baseline (speedup 1.0000x reference)
import functools

import jax
import jax.numpy as jnp
from jax import lax
from jax.experimental import pallas as pl
from jax.experimental.pallas import tpu as pltpu
from jax.experimental.pallas import tpu_sc as plsc

D = 64
S = 200
B = 4096
V = 1000000
NC = 2
NS = 16
NW = NC * NS
SCALE = 8.0

BW = B // NW
NBUF = 4
PE_S = 208
PE_D = 128


def _encoder_body(xt_hbm, tokp_hbm, pep_hbm, out_hbm,
                  xv_v, idx_v, rows_v, obuf_v, pe_v, gsem, osem, isem):
    wid = lax.axis_index("s") * NC + lax.axis_index("c")
    col0 = wid * BW

    pltpu.sync_copy(pep_hbm, pe_v)
    lanes = lax.iota(jnp.int32, 16)

    def issue_xv(c, b):
        pltpu.async_copy(xt_hbm.at[c, pl.ds(col0, BW)], xv_v.at[b],
                         isem.at[b])

    def wait_xv(b):
        pltpu.make_async_copy(xt_hbm.at[0, pl.ds(col0, BW)], xv_v.at[b],
                              isem.at[b]).wait()

    def shift_and_gather(b):
        @pl.loop(0, BW // 16)
        def _(g):
            sl = pl.ds(g * 16, 16)
            idx_v[b, sl] = lax.shift_right_logical(xv_v[b, sl], 1)
        pltpu.async_copy(tokp_hbm.at[idx_v.at[b]], rows_v.at[b], gsem.at[b])

    def wait_gather(b):
        pltpu.make_async_copy(tokp_hbm.at[idx_v.at[b]], rows_v.at[b],
                              gsem.at[b]).wait()

    def compute(c, b):
        pe_rows = [pe_v[c, pl.ds(k * 16, 16)] for k in range(D // 16)]

        @pl.loop(0, BW // 16)
        def _(g):
            sl = pl.ds(g * 16, 16)
            xg = xv_v[b, sl]
            ho = lax.shift_left(lax.bitwise_and(xg, 1), 6)
            bidx = lanes + g * 16
            for d in range(D):
                val = plsc.load_gather(rows_v.at[b], [bidx, ho + d])
                obuf_v[b, d, sl] = val * SCALE + pe_rows[d // 16][d % 16]

    def issue_scatter(c, b):
        pltpu.async_copy(obuf_v.at[b],
                         out_hbm.at[c, :, pl.ds(col0, BW)], osem.at[b])

    def wait_scatter(c, b):
        pltpu.make_async_copy(obuf_v.at[b],
                              out_hbm.at[0, :, pl.ds(col0, BW)],
                              osem.at[b]).wait()

    for b in range(NBUF - 1):
        issue_xv(b, b)
        wait_xv(b)
        shift_and_gather(b)
    issue_xv(NBUF - 1, NBUF - 1)

    @pl.loop(0, S, step=NBUF)
    def _chunks(c0):
        for b in range(NBUF):
            c = c0 + b
            prev = (b - 1) % NBUF

            @pl.when(c + NBUF - 1 < S)
            def _():
                wait_xv(prev)
                shift_and_gather(prev)

            wait_gather(b)

            @pl.when(c >= NBUF)
            def _():
                wait_scatter(c - NBUF, b)

            compute(c, b)
            issue_scatter(c, b)

            @pl.when(c + NBUF < S)
            def _():
                issue_xv(c + NBUF, b)

    for b in range(NBUF):
        wait_scatter(S - NBUF + b, (S - NBUF + b) % NBUF)


@jax.jit
def _encoder(xt, tokp, pep):
    mesh = plsc.VectorSubcoreMesh(core_axis_name="c", subcore_axis_name="s")
    return pl.kernel(
        _encoder_body,
        out_type=jax.ShapeDtypeStruct((S, D, B), jnp.float32),
        mesh=mesh,
        compiler_params=pltpu.CompilerParams(use_tc_tiling_on_sc=True, needs_layout_passes=False),
        scratch_types=[
            pltpu.VMEM((NBUF, BW), jnp.int32),
            pltpu.VMEM((NBUF, BW), jnp.int32),
            pltpu.VMEM((NBUF, BW, 128), jnp.float32),
            pltpu.VMEM((NBUF, D, BW), jnp.float32),
            pltpu.VMEM((PE_S, PE_D), jnp.float32),
            pltpu.SemaphoreType.DMA((NBUF,)),
            pltpu.SemaphoreType.DMA((NBUF,)),
            pltpu.SemaphoreType.DMA((NBUF,)),
        ],
    )(xt, tokp, pep)


def kernel(x, mask, tok_table, pos_table):
    del mask
    xt = jnp.transpose(x.astype(jnp.int32))
    tokp = jnp.reshape(tok_table, (V // 2, 128))
    pep = jnp.pad(pos_table,
                  ((0, PE_S - S), (0, PE_D - D)))
    out_t = _encoder(xt, tokp, pep)
    return jnp.transpose(out_t, (2, 0, 1))


# speedup vs baseline: 1.1141x; 1.1141x over previous
import jax
import jax.numpy as jnp
from jax import lax
from jax.experimental import pallas as pl
from jax.experimental.pallas import tpu as pltpu
from jax.experimental.pallas import tpu_sc as plsc

D = 64
S = 200
B = 4096
V = 1000000
NC = 2
NS = 16
NW = NC * NS
SCALE = 8.0

BW = B // NW
NBUF = 4
OBUF = 2
PE_S = 208
PE_D = 128
OSTR = 129


def _encoder_body(xt_hbm, tokp_hbm, pep_hbm, out_hbm,
                  xv_v, idx_v, rows_v, obuf_v, pe_v, gsem, osem, isem):
    wid = lax.axis_index("s") * NC + lax.axis_index("c")
    col0 = wid * BW

    pltpu.sync_copy(pep_hbm.at[pl.ds(0, S)], pe_v)
    lanes = lax.iota(jnp.int32, 16)
    row_ids = [lanes + 16 * k for k in range(D // 16)]

    def issue_xv(c, b):
        pltpu.async_copy(xt_hbm.at[c, pl.ds(col0, BW)], xv_v.at[b],
                         isem.at[b])

    def wait_xv(b):
        pltpu.make_async_copy(xt_hbm.at[0, pl.ds(col0, BW)], xv_v.at[b],
                              isem.at[b]).wait()

    def issue_gather(b):
        @pl.loop(0, BW // 16)
        def _(g):
            sl = pl.ds(g * 16, 16)
            idx_v[b, sl] = lax.shift_right_logical(xv_v[b, sl], 1)
        pltpu.async_copy(tokp_hbm.at[idx_v.at[b]], rows_v.at[b], gsem.at[b])

    def wait_gather(b):
        pltpu.make_async_copy(tokp_hbm.at[idx_v.at[b]], rows_v.at[b],
                              gsem.at[b]).wait()

    def compute(c, b):
        pe_ks = [pe_v[c, pl.ds(k * 16, 16)] for k in range(D // 16)]
        ob = b % OBUF

        @pl.loop(0, BW // 16)
        def _(g):
            b0 = g * 16
            xg = xv_v[b, pl.ds(b0, 16)]
            ho = lax.shift_left(lax.bitwise_and(xg, 1), 6)
            for r in range(16):
                br = b0 + r
                hr = ho[r]
                bcol = lax.broadcast(br, (16,))
                for k in range(D // 16):
                    val = rows_v[b, br, pl.ds(hr + 16 * k, 16)]
                    plsc.store_scatter(obuf_v.at[ob],
                                       [row_ids[k], bcol],
                                       val * SCALE + pe_ks[k])

    def issue_scatter(c, b):
        ob = b % OBUF
        pltpu.async_copy(obuf_v.at[ob, :, pl.ds(0, BW)],
                         out_hbm.at[c, :, pl.ds(col0, BW)], osem.at[ob])

    def wait_scatter(c, ob):
        pltpu.make_async_copy(obuf_v.at[ob, :, pl.ds(0, BW)],
                              out_hbm.at[0, :, pl.ds(col0, BW)],
                              osem.at[ob]).wait()

    for b in range(NBUF - 1):
        issue_xv(b, b)
        wait_xv(b)
        issue_gather(b)
    issue_xv(NBUF - 1, NBUF - 1)

    @pl.loop(0, S, step=NBUF)
    def _chunks(c0):
        for b in range(NBUF):
            c = c0 + b
            prev = (b - 1) % NBUF

            @pl.when(c + NBUF - 1 < S)
            def _():
                wait_xv(prev)
                issue_gather(prev)

            wait_gather(b)

            @pl.when(c >= OBUF)
            def _():
                wait_scatter(c - OBUF, b % OBUF)

            compute(c, b)
            issue_scatter(c, b)

            @pl.when(c + NBUF < S)
            def _():
                issue_xv(c + NBUF, b)

    for c in range(S - OBUF, S):
        wait_scatter(c, c % OBUF)


@jax.jit
def _encoder(xt, tokp, pep):
    mesh = plsc.VectorSubcoreMesh(core_axis_name="c", subcore_axis_name="s")
    return pl.kernel(
        _encoder_body,
        out_type=jax.ShapeDtypeStruct((S, D, B), jnp.float32),
        mesh=mesh,
        compiler_params=pltpu.CompilerParams(use_tc_tiling_on_sc=True,
                                             needs_layout_passes=False),
        scratch_types=[
            pltpu.VMEM((NBUF, BW), jnp.int32),
            pltpu.VMEM((NBUF, BW), jnp.int32),
            pltpu.VMEM((NBUF, BW, 128), jnp.float32),
            pltpu.VMEM((OBUF, D, OSTR), jnp.float32),
            pltpu.VMEM((S, PE_D), jnp.float32),
            pltpu.SemaphoreType.DMA((NBUF,)),
            pltpu.SemaphoreType.DMA((OBUF,)),
            pltpu.SemaphoreType.DMA((NBUF,)),
        ],
    )(xt, tokp, pep)


def kernel(x, mask, tok_table, pos_table):
    del mask
    xt = jnp.transpose(x.astype(jnp.int32))
    tokp = jnp.reshape(tok_table, (V // 2, 128))
    pep = jnp.pad(pos_table,
                  ((0, PE_S - S), (0, PE_D - D)))
    out_t = _encoder(xt, tokp, pep)
    return jnp.transpose(out_t, (2, 0, 1))
